# Initial kernel scaffold; baseline (speedup 1.0000x reference)
import jax
import jax.numpy as jnp
from jax.experimental import pallas as pl


def kernel(y_true, y_pred):
    raise NotImplementedError("write your pallas kernel here")


# speedup vs baseline: 1.6908x; 1.6908x over previous
import jax
import jax.numpy as jnp
from jax.experimental import pallas as pl
from jax.experimental.pallas import tpu as pltpu

_B, _H, _K = 8, 128, 128 * 64


def _body(yt0_ref, yp_ref, out_ref, acc_ref, cnt_ref):
    b = pl.program_id(0)

    s0 = yt0_ref[0]
    m1 = jnp.logical_and(s0 > 1000.0, s0 < 3000.0).astype(jnp.float32)
    m2 = jnp.logical_or(
        jnp.logical_and(s0 > 0.0, s0 < 1000.0), s0 > 3000.0
    ).astype(jnp.float32)

    yp = yp_ref[0]
    p1 = jnp.sum(yp * m1, axis=-1, keepdims=True)
    p2 = jnp.sum(yp * m2, axis=-1, keepdims=True)
    part = jnp.concatenate([p1, p2], axis=-1)
    c1 = jnp.sum(m1)
    c2 = jnp.sum(m2)

    @pl.when(b == 0)
    def _init():
        acc_ref[...] = part
        cnt_ref[0] = c1
        cnt_ref[1] = c2

    @pl.when(b > 0)
    def _accum():
        acc_ref[...] += part
        cnt_ref[0] += c1
        cnt_ref[1] += c2

    @pl.when(b == _B - 1)
    def _finish():
        acc = acc_ref[...]
        a = acc[:, 0:1] / cnt_ref[0]
        bb = acc[:, 1:2] / cnt_ref[1]
        am = a - jnp.mean(a)
        bm = bb - jnp.mean(bb)
        cov = jnp.mean(am * bm)
        sx = jnp.sqrt(jnp.mean(am * am))
        sy = jnp.sqrt(jnp.mean(bm * bm))
        corr = cov / (sx * sy)
        out_ref[...] = jnp.abs(jnp.clip(corr, -1.0, 1.0)).reshape(1, 1)


def kernel(y_true, y_pred):
    yt0 = y_true[:, 0].reshape(_B, 1, _K)
    yp = y_pred.reshape(_B, _H, _K)
    out = pl.pallas_call(
        _body,
        grid=(_B,),
        in_specs=[
            pl.BlockSpec((1, 1, _K), lambda b: (b, 0, 0)),
            pl.BlockSpec((1, _H, _K), lambda b: (b, 0, 0)),
        ],
        out_specs=pl.BlockSpec((1, 1), lambda b: (0, 0)),
        out_shape=jax.ShapeDtypeStruct((1, 1), jnp.float32),
        scratch_shapes=[
            pltpu.VMEM((_H, 2), jnp.float32),
            pltpu.SMEM((2,), jnp.float32),
        ],
    )(yt0, yp)
    return out


# speedup vs baseline: 3.3915x; 2.0058x over previous
import jax
import jax.numpy as jnp
from jax.experimental import pallas as pl
from jax.experimental.pallas import tpu as pltpu

_B, _H, _W, _D = 8, 128, 128, 64
_HC = 16
_NH = _H // _HC


def _body(yt0_ref, yp_ref, out_ref, acc1_ref, acc2_ref, cnt_ref):
    b = pl.program_id(0)
    hc = pl.program_id(1)

    s0 = yt0_ref[0, 0]
    m1 = jnp.logical_and(s0 > 1000.0, s0 < 3000.0).astype(jnp.float32)
    m2 = jnp.logical_or(
        jnp.logical_and(s0 > 0.0, s0 < 1000.0), s0 > 3000.0
    ).astype(jnp.float32)

    yp = yp_ref[0]
    p1 = jnp.sum(yp * m1[None], axis=1)
    p2 = jnp.sum(yp * m2[None], axis=1)

    sl = pl.ds(hc * _HC, _HC)

    @pl.when(b == 0)
    def _init():
        acc1_ref[sl, :] = p1
        acc2_ref[sl, :] = p2

    @pl.when(b > 0)
    def _accum():
        acc1_ref[sl, :] += p1
        acc2_ref[sl, :] += p2

    @pl.when(hc == 0)
    def _counts():
        c1 = jnp.sum(m1)
        c2 = jnp.sum(m2)

        @pl.when(b == 0)
        def _cinit():
            cnt_ref[0] = c1
            cnt_ref[1] = c2

        @pl.when(b > 0)
        def _cacc():
            cnt_ref[0] += c1
            cnt_ref[1] += c2

    @pl.when(jnp.logical_and(b == _B - 1, hc == _NH - 1))
    def _finish():
        a = jnp.sum(acc1_ref[...], axis=1, keepdims=True) / cnt_ref[0]
        bb = jnp.sum(acc2_ref[...], axis=1, keepdims=True) / cnt_ref[1]
        am = a - jnp.mean(a)
        bm = bb - jnp.mean(bb)
        cov = jnp.mean(am * bm)
        sx = jnp.sqrt(jnp.mean(am * am))
        sy = jnp.sqrt(jnp.mean(bm * bm))
        corr = cov / (sx * sy)
        out_ref[...] = jnp.abs(jnp.clip(corr, -1.0, 1.0)).reshape(1, 1)


def kernel(y_true, y_pred):
    yt = jnp.transpose(y_true[..., 0], (0, 1, 3, 2))
    yp = jnp.transpose(y_pred[..., 0], (0, 1, 3, 2))
    out = pl.pallas_call(
        _body,
        grid=(_B, _NH),
        in_specs=[
            pl.BlockSpec((1, 1, _D, _W), lambda b, hc: (b, 0, 0, 0)),
            pl.BlockSpec((1, _HC, _D, _W), lambda b, hc: (b, hc, 0, 0)),
        ],
        out_specs=pl.BlockSpec((1, 1), lambda b, hc: (0, 0)),
        out_shape=jax.ShapeDtypeStruct((1, 1), jnp.float32),
        scratch_shapes=[
            pltpu.VMEM((_H, _W), jnp.float32),
            pltpu.VMEM((_H, _W), jnp.float32),
            pltpu.SMEM((2,), jnp.float32),
        ],
    )(yt, yp)
    return out
